# baseline (speedup 1.0000x reference)
import functools

import jax
import jax.numpy as jnp
from jax import lax
from jax.experimental import pallas as pl
from jax.experimental.pallas import tpu as pltpu
from jax.experimental.pallas import tpu_sc as plsc

N = 40000
NP = 40960
C = 128
MH = 8
P = 4
D = 16
GRID = 200

NBA = 512
NBC = 1000

NC = 2
NS = 16
NW = NC * NS
CH = 16
CHUNKS = NP // (CH * NW)

_PREC = jax.lax.Precision.HIGHEST


def _stage_a_body(q_ref, wv_ref, bv_ref, wa_ref, ba_ref, wo_ref, bo_ref,
                  val_ref, idx_ref, w_ref):
    bid = pl.program_id(0)
    q = q_ref[...]
    MP = MH * P

    val = jnp.dot(q, wv_ref[...], precision=_PREC,
                  preferred_element_type=jnp.float32) + bv_ref[...]
    val_ref[...] = val

    a = jnp.dot(q, wa_ref[...], precision=_PREC,
                preferred_element_type=jnp.float32) + ba_ref[...]
    e = jnp.exp(jnp.clip(a, -80.0, 80.0))
    gi = lax.broadcasted_iota(jnp.int32, (MP, MP), 0) // P
    gj = lax.broadcasted_iota(jnp.int32, (MP, MP), 1) // P
    gmat = (gi == gj).astype(jnp.float32)
    s = jnp.dot(e, gmat, precision=_PREC, preferred_element_type=jnp.float32)
    attn = e / s

    o = jnp.dot(q, wo_ref[...], precision=_PREC,
                preferred_element_type=jnp.float32) + bo_ref[...]
    offx = o[:, :MP]
    offy = o[:, MP:]

    n = bid * NBA + lax.broadcasted_iota(jnp.int32, (NBA, MP), 0)
    i = n // GRID
    j = n - i * GRID
    x = i.astype(jnp.float32) + offx
    y = j.astype(jnp.float32) + offy
    x0 = jnp.clip(jnp.floor(x), -2.0, 200.0)
    y0 = jnp.clip(jnp.floor(y), -2.0, 200.0)
    fx = x - x0
    fy = y - y0
    xb = jnp.clip(x0, 0.0, 198.0)
    yb = jnp.clip(y0, 0.0, 198.0)
    dx = x0 - xb
    dy = y0 - yb
    zero = jnp.zeros_like(fx)
    xl0 = jnp.where(dx == 0.0, 1.0 - fx, jnp.where(dx == -1.0, fx, zero))
    xl1 = jnp.where(dx == 0.0, fx, jnp.where(dx == 1.0, 1.0 - fx, zero))
    yl0 = jnp.where(dy == 0.0, 1.0 - fy, jnp.where(dy == -1.0, fy, zero))
    yl1 = jnp.where(dy == 0.0, fy, jnp.where(dy == 1.0, 1.0 - fy, zero))

    m_f = (lax.broadcasted_iota(jnp.int32, (NBA, MP), 1) // P).astype(jnp.float32)
    base = m_f * float(NP) + yb * float(GRID) + xb

    iy0 = base
    iy1 = base + float(GRID)
    w00 = attn * yl0 * xl0
    w01 = attn * yl0 * xl1
    w10 = attn * yl1 * xl0
    w11 = attn * yl1 * xl1

    ci = lax.broadcasted_iota(jnp.int32, (C, C), 0)
    cj = lax.broadcasted_iota(jnp.int32, (C, C), 1)
    pmat = (cj == (ci % MP) * 4 + ci // MP).astype(jnp.float32)
    wf = jnp.concatenate([w00, w01, w10, w11], axis=1)
    wf = jnp.dot(wf, pmat, precision=_PREC, preferred_element_type=jnp.float32)
    di = lax.broadcasted_iota(jnp.int32, (2 * MP, 2 * MP), 0)
    dj = lax.broadcasted_iota(jnp.int32, (2 * MP, 2 * MP), 1)
    pmat2 = (dj == (di % MP) * 2 + di // MP).astype(jnp.float32)
    idxf = jnp.concatenate([iy0, iy1], axis=1)
    idxf = jnp.dot(idxf, pmat2, precision=_PREC, preferred_element_type=jnp.float32)

    idx_ref[...] = jnp.clip(idxf.astype(jnp.int32), 0, MH * NP - 1)
    w_ref[...] = wf


def _stage_a(q, wv, bv, wa, ba, wo, bo):
    full = lambda s: pl.BlockSpec(s, lambda b: (0,) * len(s))
    return pl.pallas_call(
        _stage_a_body,
        grid=(NP // NBA,),
        in_specs=[
            pl.BlockSpec((NBA, C), lambda b: (b, 0)),
            full((C, C)), full((C,)),
            full((C, MH * P)), full((MH * P,)),
            full((C, 2 * MH * P)), full((2 * MH * P,)),
        ],
        out_specs=[
            pl.BlockSpec((NBA, C), lambda b: (b, 0)),
            pl.BlockSpec((NBA, C // 2), lambda b: (b, 0)),
            pl.BlockSpec((NBA, C), lambda b: (b, 0)),
        ],
        out_shape=[
            jax.ShapeDtypeStruct((NP, C), jnp.float32),
            jax.ShapeDtypeStruct((NP, C // 2), jnp.int32),
            jax.ShapeDtypeStruct((NP, C), jnp.float32),
        ],
    )(q, wv, bv, wa, ba, wo, bo)


def _sc_body(table_hbm, idx_hbm, w_hbm, out_hbm,
             idx0, idx1, w0, w1, rows0, rows1, out0, out1,
             semg0, semg1, semiw0, semiw1, semo0, semo1):
    wid = lax.axis_index("s") * NC + lax.axis_index("c")

    def fire_iw(c, idxb, wb, sem):
        base = (wid + c * NW) * CH
        pltpu.async_copy(idx_hbm.at[pl.ds(base, CH)], idxb, sem)
        pltpu.async_copy(w_hbm.at[pl.ds(base, CH)], wb, sem)

    def wait_iw(c, idxb, wb, sem):
        base = (wid + c * NW) * CH
        pltpu.make_async_copy(idx_hbm.at[pl.ds(base, CH)], idxb, sem).wait()
        pltpu.make_async_copy(w_hbm.at[pl.ds(base, CH)], wb, sem).wait()

    def fire_g(idxb, rowsb, sem):
        for r in range(CH):
            pltpu.async_copy(table_hbm.at[idxb.at[r]], rowsb.at[r], sem)

    def drain_g(idxb, rowsb, sem):
        for r in range(CH):
            pltpu.make_async_copy(table_hbm.at[idxb.at[r]], rowsb.at[r], sem).wait()

    def compute(wb, rowsb, outb):
        def row(r, carry):
            for m in range(MH):
                wv16 = wb[r, pl.ds(m * D, D)]
                acc = jnp.zeros((D,), jnp.float32)
                for p in range(P):
                    for cy in range(2):
                        pr = (m * P + p) * 2 + cy
                        acc = acc + wv16[p * 4 + cy * 2] * rowsb[r, pr, pl.ds(0, D)]
                        acc = acc + wv16[p * 4 + cy * 2 + 1] * rowsb[r, pr, pl.ds(D, D)]
                outb[r, pl.ds(m * D, D)] = acc
            return carry

        lax.fori_loop(0, CH, row, 0)

    def fire_out(c, outb, sem):
        base = (wid + c * NW) * CH
        pltpu.async_copy(outb, out_hbm.at[pl.ds(base, CH)], sem)

    def wait_out(c, outb, sem):
        base = (wid + c * NW) * CH
        pltpu.make_async_copy(outb, out_hbm.at[pl.ds(base, CH)], sem).wait()

    fire_iw(0, idx0, w0, semiw0)
    wait_iw(0, idx0, w0, semiw0)
    fire_g(idx0, rows0, semg0)
    fire_iw(1, idx1, w1, semiw1)
    wait_iw(1, idx1, w1, semiw1)
    fire_g(idx1, rows1, semg1)
    drain_g(idx0, rows0, semg0)
    compute(w0, rows0, out0)
    fire_out(0, out0, semo0)
    fire_iw(2, idx0, w0, semiw0)
    wait_iw(2, idx0, w0, semiw0)
    fire_g(idx0, rows0, semg0)
    drain_g(idx1, rows1, semg1)
    compute(w1, rows1, out1)
    fire_out(1, out1, semo1)
    fire_iw(3, idx1, w1, semiw1)

    def pair(i, carry):
        c0 = 2 * i
        wait_iw(c0 + 1, idx1, w1, semiw1)
        fire_g(idx1, rows1, semg1)
        drain_g(idx0, rows0, semg0)
        wait_out(c0 - 2, out0, semo0)
        compute(w0, rows0, out0)
        fire_out(c0, out0, semo0)
        fire_iw(c0 + 2, idx0, w0, semiw0)
        wait_iw(c0 + 2, idx0, w0, semiw0)
        fire_g(idx0, rows0, semg0)
        drain_g(idx1, rows1, semg1)
        wait_out(c0 - 1, out1, semo1)
        compute(w1, rows1, out1)
        fire_out(c0 + 1, out1, semo1)
        fire_iw(c0 + 3, idx1, w1, semiw1)
        return carry

    lax.fori_loop(1, CHUNKS // 2 - 1, pair, 0)
    c0 = CHUNKS - 2
    wait_iw(c0 + 1, idx1, w1, semiw1)
    fire_g(idx1, rows1, semg1)
    drain_g(idx0, rows0, semg0)
    wait_out(c0 - 2, out0, semo0)
    compute(w0, rows0, out0)
    fire_out(c0, out0, semo0)
    drain_g(idx1, rows1, semg1)
    wait_out(c0 - 1, out1, semo1)
    compute(w1, rows1, out1)
    fire_out(c0 + 1, out1, semo1)
    wait_out(c0, out0, semo0)
    wait_out(c0 + 1, out1, semo1)


def _stage_b(table, idx, w):
    mesh = plsc.VectorSubcoreMesh(core_axis_name="c", subcore_axis_name="s")
    f = functools.partial(
        pl.kernel,
        mesh=mesh,
        compiler_params=pltpu.CompilerParams(use_tc_tiling_on_sc=False),
        out_type=jax.ShapeDtypeStruct((NP, C), jnp.float32),
        scratch_types=[
            pltpu.VMEM((CH, C // 2), jnp.int32),
            pltpu.VMEM((CH, C // 2), jnp.int32),
            pltpu.VMEM((CH, C), jnp.float32),
            pltpu.VMEM((CH, C), jnp.float32),
            pltpu.VMEM((CH, C // 2, 2 * D), jnp.float32),
            pltpu.VMEM((CH, C // 2, 2 * D), jnp.float32),
            pltpu.VMEM((CH, C), jnp.float32),
            pltpu.VMEM((CH, C), jnp.float32),
            pltpu.SemaphoreType.DMA,
            pltpu.SemaphoreType.DMA,
            pltpu.SemaphoreType.DMA,
            pltpu.SemaphoreType.DMA,
            pltpu.SemaphoreType.DMA,
            pltpu.SemaphoreType.DMA,
        ],
    )(_sc_body)
    return f(table, idx, w)


def _stage_c_body(s_ref, q_ref, w_ref, b_ref, out_ref):
    s = s_ref[...]
    out = jnp.dot(s, w_ref[...], precision=_PREC,
                  preferred_element_type=jnp.float32) + b_ref[...]
    out_ref[...] = out + q_ref[...]


def _stage_c(sampled, q, w_out, b_out):
    full = lambda s: pl.BlockSpec(s, lambda b: (0,) * len(s))
    return pl.pallas_call(
        _stage_c_body,
        grid=(N // NBC,),
        in_specs=[
            pl.BlockSpec((NBC, C), lambda b: (b, 0)),
            pl.BlockSpec((NBC, C), lambda b: (b, 0)),
            full((C, C)), full((C,)),
        ],
        out_specs=pl.BlockSpec((NBC, C), lambda b: (b, 0)),
        out_shape=jax.ShapeDtypeStruct((N, C), jnp.float32),
    )(sampled, q, w_out, b_out)


def kernel(query, W_off, b_off, W_attn, b_attn, W_val, b_val, W_out, b_out):
    q = query.reshape(N, C)
    qp = jnp.pad(q, ((0, NP - N), (0, 0)))
    perm = jnp.concatenate([jnp.arange(MH * P) * 2, jnp.arange(MH * P) * 2 + 1])
    wo = W_off[:, perm]
    bo = b_off[perm]

    val, idx, w = _stage_a(qp, W_val, b_val, W_attn, b_attn, wo, bo)
    table1 = jnp.concatenate(
        [val[:, m * D:(m + 1) * D] for m in range(MH)], axis=0)
    table = jnp.concatenate(
        [table1, jnp.roll(table1, -1, axis=0)], axis=1)
    sampled = _stage_b(table, idx, w)
    out = _stage_c(sampled, q, W_out, b_out)
    return out.reshape(1, N, C)


# speedup vs baseline: 1.3244x; 1.3244x over previous
import functools

import jax
import jax.numpy as jnp
from jax import lax
from jax.experimental import pallas as pl
from jax.experimental.pallas import tpu as pltpu
from jax.experimental.pallas import tpu_sc as plsc

N = 40000
NP = 40960
C = 128
MH = 8
P = 4
D = 16
GRID = 200

NBA = 512
NBC = 1000

NC = 2
NS = 16
NW = NC * NS
CH = 16
CHUNKS = NP // (CH * NW)

_PREC = jax.lax.Precision.HIGHEST


def _stage_a_body(q_ref, wv_ref, bv_ref, wa_ref, ba_ref, wo_ref, bo_ref,
                  val_ref, idx_ref, w_ref):
    bid = pl.program_id(0)
    q = q_ref[...]
    MP = MH * P

    val = jnp.dot(q, wv_ref[...],
                  preferred_element_type=jnp.float32) + bv_ref[...]
    val_ref[...] = val

    a = jnp.dot(q, wa_ref[...],
                preferred_element_type=jnp.float32) + ba_ref[...]
    e = jnp.exp(jnp.clip(a, -80.0, 80.0))
    gi = lax.broadcasted_iota(jnp.int32, (MP, MP), 0) // P
    gj = lax.broadcasted_iota(jnp.int32, (MP, MP), 1) // P
    gmat = (gi == gj).astype(jnp.float32)
    s = jnp.dot(e, gmat, precision=_PREC, preferred_element_type=jnp.float32)
    attn = e / s

    o = jnp.dot(q, wo_ref[...],
                preferred_element_type=jnp.float32) + bo_ref[...]
    offx = o[:, :MP]
    offy = o[:, MP:]

    n = bid * NBA + lax.broadcasted_iota(jnp.int32, (NBA, MP), 0)
    i = n // GRID
    j = n - i * GRID
    x = i.astype(jnp.float32) + offx
    y = j.astype(jnp.float32) + offy
    x0 = jnp.clip(jnp.floor(x), -2.0, 200.0)
    y0 = jnp.clip(jnp.floor(y), -2.0, 200.0)
    fx = x - x0
    fy = y - y0
    xb = jnp.clip(x0, 0.0, 198.0)
    yb = jnp.clip(y0, 0.0, 198.0)
    dx = x0 - xb
    dy = y0 - yb
    zero = jnp.zeros_like(fx)
    xl0 = jnp.where(dx == 0.0, 1.0 - fx, jnp.where(dx == -1.0, fx, zero))
    xl1 = jnp.where(dx == 0.0, fx, jnp.where(dx == 1.0, 1.0 - fx, zero))
    yl0 = jnp.where(dy == 0.0, 1.0 - fy, jnp.where(dy == -1.0, fy, zero))
    yl1 = jnp.where(dy == 0.0, fy, jnp.where(dy == 1.0, 1.0 - fy, zero))

    m_f = (lax.broadcasted_iota(jnp.int32, (NBA, MP), 1) // P).astype(jnp.float32)
    base = m_f * float(NP) + yb * float(GRID) + xb

    i00 = base
    i01 = base + 1.0
    i10 = base + float(GRID)
    i11 = base + float(GRID) + 1.0
    w00 = attn * yl0 * xl0
    w01 = attn * yl0 * xl1
    w10 = attn * yl1 * xl0
    w11 = attn * yl1 * xl1

    ci = lax.broadcasted_iota(jnp.int32, (C, C), 0)
    cj = lax.broadcasted_iota(jnp.int32, (C, C), 1)
    pmat = (cj == (ci % MP) * 4 + ci // MP).astype(jnp.float32)
    idxf = jnp.concatenate([i00, i01, i10, i11], axis=1)
    wf = jnp.concatenate([w00, w01, w10, w11], axis=1)
    idxf = jnp.dot(idxf, pmat, precision=_PREC, preferred_element_type=jnp.float32)
    wf = jnp.dot(wf, pmat, precision=_PREC, preferred_element_type=jnp.float32)

    idx_ref[...] = jnp.clip(idxf.astype(jnp.int32), 0, MH * NP - 1)
    w_ref[...] = wf


def _stage_a(q, wv, bv, wa, ba, wo, bo):
    full = lambda s: pl.BlockSpec(s, lambda b: (0,) * len(s))
    return pl.pallas_call(
        _stage_a_body,
        grid=(NP // NBA,),
        in_specs=[
            pl.BlockSpec((NBA, C), lambda b: (b, 0)),
            full((C, C)), full((C,)),
            full((C, MH * P)), full((MH * P,)),
            full((C, 2 * MH * P)), full((2 * MH * P,)),
        ],
        out_specs=[
            pl.BlockSpec((NBA, C), lambda b: (b, 0)),
            pl.BlockSpec((NBA, C), lambda b: (b, 0)),
            pl.BlockSpec((NBA, C), lambda b: (b, 0)),
        ],
        out_shape=[
            jax.ShapeDtypeStruct((NP, C), jnp.float32),
            jax.ShapeDtypeStruct((NP, C), jnp.int32),
            jax.ShapeDtypeStruct((NP, C), jnp.float32),
        ],
    )(q, wv, bv, wa, ba, wo, bo)


def _sc_body(table_hbm, idx_hbm, w_hbm, out_hbm,
             idx0, idx1, w0, w1, rows0, rows1, out0, out1,
             semg0, semg1, semiw0, semiw1, semo0, semo1):
    wid = lax.axis_index("s") * NC + lax.axis_index("c")

    def fire_iw(c, idxb, wb, sem):
        base = (wid + c * NW) * CH
        pltpu.async_copy(idx_hbm.at[pl.ds(base, CH)], idxb, sem)
        pltpu.async_copy(w_hbm.at[pl.ds(base, CH)], wb, sem)

    def wait_iw(c, idxb, wb, sem):
        base = (wid + c * NW) * CH
        pltpu.make_async_copy(idx_hbm.at[pl.ds(base, CH)], idxb, sem).wait()
        pltpu.make_async_copy(w_hbm.at[pl.ds(base, CH)], wb, sem).wait()

    def fire_g(idxb, rowsb, sem):
        for r in range(CH):
            pltpu.async_copy(table_hbm.at[idxb.at[r]], rowsb.at[r], sem)

    def drain_g(idxb, rowsb, sem):
        for r in range(CH):
            pltpu.make_async_copy(table_hbm.at[idxb.at[r]], rowsb.at[r], sem).wait()

    def compute(wb, rowsb, outb):
        def row(r, carry):
            for m in range(MH):
                wv16 = wb[r, pl.ds(m * D, D)]
                acc = jnp.zeros((D,), jnp.float32)
                for k in range(16):
                    acc = acc + wv16[k] * rowsb[r, m * D + k, :]
                outb[r, pl.ds(m * D, D)] = acc
            return carry

        lax.fori_loop(0, CH, row, 0)

    def fire_out(c, outb, sem):
        base = (wid + c * NW) * CH
        pltpu.async_copy(outb, out_hbm.at[pl.ds(base, CH)], sem)

    def wait_out(c, outb, sem):
        base = (wid + c * NW) * CH
        pltpu.make_async_copy(outb, out_hbm.at[pl.ds(base, CH)], sem).wait()

    fire_iw(0, idx0, w0, semiw0)
    wait_iw(0, idx0, w0, semiw0)
    fire_g(idx0, rows0, semg0)
    fire_iw(1, idx1, w1, semiw1)
    wait_iw(1, idx1, w1, semiw1)
    fire_g(idx1, rows1, semg1)
    drain_g(idx0, rows0, semg0)
    compute(w0, rows0, out0)
    fire_out(0, out0, semo0)
    fire_iw(2, idx0, w0, semiw0)
    wait_iw(2, idx0, w0, semiw0)
    fire_g(idx0, rows0, semg0)
    drain_g(idx1, rows1, semg1)
    compute(w1, rows1, out1)
    fire_out(1, out1, semo1)
    fire_iw(3, idx1, w1, semiw1)

    def pair(i, carry):
        c0 = 2 * i
        wait_iw(c0 + 1, idx1, w1, semiw1)
        fire_g(idx1, rows1, semg1)
        drain_g(idx0, rows0, semg0)
        wait_out(c0 - 2, out0, semo0)
        compute(w0, rows0, out0)
        fire_out(c0, out0, semo0)
        fire_iw(c0 + 2, idx0, w0, semiw0)
        wait_iw(c0 + 2, idx0, w0, semiw0)
        fire_g(idx0, rows0, semg0)
        drain_g(idx1, rows1, semg1)
        wait_out(c0 - 1, out1, semo1)
        compute(w1, rows1, out1)
        fire_out(c0 + 1, out1, semo1)
        fire_iw(c0 + 3, idx1, w1, semiw1)
        return carry

    lax.fori_loop(1, CHUNKS // 2 - 1, pair, 0)
    c0 = CHUNKS - 2
    wait_iw(c0 + 1, idx1, w1, semiw1)
    fire_g(idx1, rows1, semg1)
    drain_g(idx0, rows0, semg0)
    wait_out(c0 - 2, out0, semo0)
    compute(w0, rows0, out0)
    fire_out(c0, out0, semo0)
    drain_g(idx1, rows1, semg1)
    wait_out(c0 - 1, out1, semo1)
    compute(w1, rows1, out1)
    fire_out(c0 + 1, out1, semo1)
    wait_out(c0, out0, semo0)
    wait_out(c0 + 1, out1, semo1)


def _stage_b(table, idx, w):
    mesh = plsc.VectorSubcoreMesh(core_axis_name="c", subcore_axis_name="s")
    f = functools.partial(
        pl.kernel,
        mesh=mesh,
        compiler_params=pltpu.CompilerParams(use_tc_tiling_on_sc=False),
        out_type=jax.ShapeDtypeStruct((NP, C), jnp.float32),
        scratch_types=[
            pltpu.VMEM((CH, C), jnp.int32),
            pltpu.VMEM((CH, C), jnp.int32),
            pltpu.VMEM((CH, C), jnp.float32),
            pltpu.VMEM((CH, C), jnp.float32),
            pltpu.VMEM((CH, C, D), jnp.float32),
            pltpu.VMEM((CH, C, D), jnp.float32),
            pltpu.VMEM((CH, C), jnp.float32),
            pltpu.VMEM((CH, C), jnp.float32),
            pltpu.SemaphoreType.DMA,
            pltpu.SemaphoreType.DMA,
            pltpu.SemaphoreType.DMA,
            pltpu.SemaphoreType.DMA,
            pltpu.SemaphoreType.DMA,
            pltpu.SemaphoreType.DMA,
        ],
    )(_sc_body)
    return f(table, idx, w)


def _stage_c_body(s_ref, q_ref, w_ref, b_ref, out_ref):
    s = s_ref[...]
    out = jnp.dot(s, w_ref[...],
                  preferred_element_type=jnp.float32) + b_ref[...]
    out_ref[...] = out + q_ref[...]


def _stage_c(sampled, q, w_out, b_out):
    full = lambda s: pl.BlockSpec(s, lambda b: (0,) * len(s))
    return pl.pallas_call(
        _stage_c_body,
        grid=(N // NBC,),
        in_specs=[
            pl.BlockSpec((NBC, C), lambda b: (b, 0)),
            pl.BlockSpec((NBC, C), lambda b: (b, 0)),
            full((C, C)), full((C,)),
        ],
        out_specs=pl.BlockSpec((NBC, C), lambda b: (b, 0)),
        out_shape=jax.ShapeDtypeStruct((N, C), jnp.float32),
    )(sampled, q, w_out, b_out)


def kernel(query, W_off, b_off, W_attn, b_attn, W_val, b_val, W_out, b_out):
    q = query.reshape(N, C)
    qp = jnp.pad(q, ((0, NP - N), (0, 0)))
    perm = jnp.concatenate([jnp.arange(MH * P) * 2, jnp.arange(MH * P) * 2 + 1])
    wo = W_off[:, perm]
    bo = b_off[perm]

    val, idx, w = _stage_a(qp, W_val, b_val, W_attn, b_attn, wo, bo)
    table = jnp.concatenate(
        [val[:, m * D:(m + 1) * D] for m in range(MH)], axis=0)
    sampled = _stage_b(table, idx, w)
    out = _stage_c(sampled, q, W_out, b_out)
    return out.reshape(1, N, C)
